# baseline (speedup 1.0000x reference)
import functools
import math

import jax
import jax.numpy as jnp
from jax import lax
from jax.experimental import pallas as pl
from jax.experimental.pallas import tpu as pltpu
from jax.experimental.pallas import tpu_sc as plsc

SMOOTHING = 0.1
CONFIDENCE = 1.0 - SMOOTHING
_BLK = 1000

_NC = 2
_NS = 16
_NW = _NC * _NS


def _sc_gather(xt, target, n):
    bpw = n // _NW
    mesh = plsc.VectorSubcoreMesh(core_axis_name="c", subcore_axis_name="s")

    @functools.partial(
        pl.kernel,
        out_type=jax.ShapeDtypeStruct((n,), jnp.float32),
        mesh=mesh,
        scratch_types=[
            pltpu.VMEM((bpw,), jnp.int32),
            pltpu.VMEM((bpw, n), jnp.float32),
            pltpu.VMEM((bpw,), jnp.float32),
            pltpu.SemaphoreType.DMA,
        ],
        compiler_params=pltpu.CompilerParams(use_tc_tiling_on_sc=True),
    )
    def gk(xt_hbm, tgt_hbm, out_hbm, idx_v, rows_v, vals_v, sem):
        wid = lax.axis_index("s") * _NC + lax.axis_index("c")
        base = wid * bpw
        pltpu.sync_copy(tgt_hbm.at[pl.ds(base, bpw)], idx_v)
        pltpu.async_copy(xt_hbm.at[idx_v], rows_v, sem).wait()
        for c in range(bpw // 16):
            acc = jnp.zeros((16,), jnp.float32)
            win = base + c * 16
            for k in range(16):
                j = c * 16 + k
                vj = rows_v[j, pl.ds(win, 16)]
                acc = jnp.where(lax.iota(jnp.int32, 16) == k, vj, acc)
            vals_v[pl.ds(c * 16, 16)] = acc
        pltpu.sync_copy(vals_v, out_hbm.at[pl.ds(base, bpw)])

    return gk(xt, target)


def _main_body(xt_ref, o_ref):
    i = pl.program_id(0)

    @pl.when(i == 0)
    def _():
        o_ref[0, 0] = jnp.float32(0.0)

    o_ref[0, 0] += jnp.sum(jnp.log2(xt_ref[...]))


def _combine_body(g_ref, s_ref, o_ref, *, const_term, eps):
    ln2 = math.log(2.0)
    s_tgt = jnp.sum(jnp.log2(g_ref[...]))
    o_ref[0, 0] = (jnp.float32(const_term)
                   + jnp.float32(-eps * ln2) * s_ref[0, 0]
                   + jnp.float32((eps - CONFIDENCE) * ln2) * s_tgt)


def kernel(x, target):
    n, size = x.shape
    eps = SMOOTHING / (size - 1)
    const_term = n * ((size - 1) * eps * math.log(eps)
                      + CONFIDENCE * math.log(CONFIDENCE))

    xt = x.T
    g = _sc_gather(xt, target, n)

    s_all = pl.pallas_call(
        _main_body,
        grid=(size // _BLK,),
        in_specs=[pl.BlockSpec((_BLK, n), lambda i: (i, 0))],
        out_specs=pl.BlockSpec(memory_space=pltpu.SMEM),
        out_shape=jax.ShapeDtypeStruct((1, 1), jnp.float32),
        compiler_params=pltpu.CompilerParams(
            dimension_semantics=("arbitrary",),
        ),
    )(xt)

    combine = functools.partial(_combine_body, const_term=const_term, eps=eps)
    out = pl.pallas_call(
        combine,
        in_specs=[
            pl.BlockSpec((8, n // 8), lambda: (0, 0)),
            pl.BlockSpec(memory_space=pltpu.SMEM),
        ],
        out_specs=pl.BlockSpec(memory_space=pltpu.SMEM),
        out_shape=jax.ShapeDtypeStruct((1, 1), jnp.float32),
    )(g.reshape(8, n // 8), s_all)
    return out[0, 0]


# speedup vs baseline: 1.2900x; 1.2900x over previous
import functools
import math

import jax
import jax.numpy as jnp
from jax import lax
from jax.experimental import pallas as pl
from jax.experimental.pallas import tpu as pltpu
from jax.experimental.pallas import tpu_sc as plsc

SMOOTHING = 0.1
CONFIDENCE = 1.0 - SMOOTHING
_BLK = 4000

_NC = 2
_NS = 16
_NW = _NC * _NS


def _sc_gather(xt, target, n):
    bpw = n // _NW
    mesh = plsc.VectorSubcoreMesh(core_axis_name="c", subcore_axis_name="s")

    @functools.partial(
        pl.kernel,
        out_type=jax.ShapeDtypeStruct((n,), jnp.float32),
        mesh=mesh,
        scratch_types=[
            pltpu.VMEM((bpw,), jnp.int32),
            pltpu.VMEM((bpw, n), jnp.float32),
            pltpu.VMEM((bpw,), jnp.float32),
            pltpu.SemaphoreType.DMA,
        ],
        compiler_params=pltpu.CompilerParams(use_tc_tiling_on_sc=True),
    )
    def gk(xt_hbm, tgt_hbm, out_hbm, idx_v, rows_v, vals_v, sem):
        wid = lax.axis_index("s") * _NC + lax.axis_index("c")
        base = wid * bpw
        pltpu.sync_copy(tgt_hbm.at[pl.ds(base, bpw)], idx_v)
        pltpu.async_copy(xt_hbm.at[idx_v], rows_v, sem).wait()
        for c in range(bpw // 16):
            acc = jnp.zeros((16,), jnp.float32)
            win = base + c * 16
            for k in range(16):
                j = c * 16 + k
                vj = rows_v[j, pl.ds(win, 16)]
                acc = jnp.where(lax.iota(jnp.int32, 16) == k, vj, acc)
            vals_v[pl.ds(c * 16, 16)] = acc
        pltpu.sync_copy(vals_v, out_hbm.at[pl.ds(base, bpw)])

    return gk(xt, target)


def _main_body(xt_ref, o_ref):
    i = pl.program_id(0)

    @pl.when(i == 0)
    def _():
        o_ref[0, 0] = jnp.float32(0.0)

    o_ref[0, 0] += jnp.sum(jnp.log2(xt_ref[...]))


def _combine_body(g_ref, s_ref, o_ref, *, const_term, eps):
    ln2 = math.log(2.0)
    s_tgt = jnp.sum(jnp.log2(g_ref[...]))
    o_ref[0, 0] = (jnp.float32(const_term)
                   + jnp.float32(-eps * ln2) * s_ref[0, 0]
                   + jnp.float32((eps - CONFIDENCE) * ln2) * s_tgt)


def kernel(x, target):
    n, size = x.shape
    eps = SMOOTHING / (size - 1)
    const_term = n * ((size - 1) * eps * math.log(eps)
                      + CONFIDENCE * math.log(CONFIDENCE))

    xt = x.T
    g = _sc_gather(xt, target, n)

    s_all = pl.pallas_call(
        _main_body,
        grid=(size // _BLK,),
        in_specs=[pl.BlockSpec((_BLK, n), lambda i: (i, 0))],
        out_specs=pl.BlockSpec(memory_space=pltpu.SMEM),
        out_shape=jax.ShapeDtypeStruct((1, 1), jnp.float32),
        compiler_params=pltpu.CompilerParams(
            dimension_semantics=("arbitrary",),
        ),
    )(xt)

    combine = functools.partial(_combine_body, const_term=const_term, eps=eps)
    out = pl.pallas_call(
        combine,
        in_specs=[
            pl.BlockSpec((8, n // 8), lambda: (0, 0)),
            pl.BlockSpec(memory_space=pltpu.SMEM),
        ],
        out_specs=pl.BlockSpec(memory_space=pltpu.SMEM),
        out_shape=jax.ShapeDtypeStruct((1, 1), jnp.float32),
    )(g.reshape(8, n // 8), s_all)
    return out[0, 0]


# speedup vs baseline: 1.2973x; 1.0057x over previous
import functools
import math

import jax
import jax.numpy as jnp
from jax import lax
from jax.experimental import pallas as pl
from jax.experimental.pallas import tpu as pltpu
from jax.experimental.pallas import tpu_sc as plsc

SMOOTHING = 0.1
CONFIDENCE = 1.0 - SMOOTHING
_BLK = 5000

_NC = 2
_NS = 16
_NW = _NC * _NS


def _sc_gather(xt, target, n):
    bpw = n // _NW
    mesh = plsc.VectorSubcoreMesh(core_axis_name="c", subcore_axis_name="s")

    @functools.partial(
        pl.kernel,
        out_type=jax.ShapeDtypeStruct((n,), jnp.float32),
        mesh=mesh,
        scratch_types=[
            pltpu.VMEM((bpw,), jnp.int32),
            pltpu.VMEM((bpw, n), jnp.float32),
            pltpu.VMEM((bpw,), jnp.float32),
            pltpu.SemaphoreType.DMA,
        ],
        compiler_params=pltpu.CompilerParams(use_tc_tiling_on_sc=True),
    )
    def gk(xt_hbm, tgt_hbm, out_hbm, idx_v, rows_v, vals_v, sem):
        wid = lax.axis_index("s") * _NC + lax.axis_index("c")
        base = wid * bpw
        pltpu.sync_copy(tgt_hbm.at[pl.ds(base, bpw)], idx_v)
        pltpu.async_copy(xt_hbm.at[idx_v], rows_v, sem).wait()
        for c in range(bpw // 16):
            acc = jnp.zeros((16,), jnp.float32)
            win = base + c * 16
            for k in range(16):
                j = c * 16 + k
                vj = rows_v[j, pl.ds(win, 16)]
                acc = jnp.where(lax.iota(jnp.int32, 16) == k, vj, acc)
            vals_v[pl.ds(c * 16, 16)] = acc
        pltpu.sync_copy(vals_v, out_hbm.at[pl.ds(base, bpw)])

    return gk(xt, target)


def _main_body(xt_ref, o_ref):
    i = pl.program_id(0)

    @pl.when(i == 0)
    def _():
        o_ref[0, 0] = jnp.float32(0.0)

    o_ref[0, 0] += jnp.sum(jnp.log2(xt_ref[...]))


def _combine_body(g_ref, s_ref, o_ref, *, const_term, eps):
    ln2 = math.log(2.0)
    s_tgt = jnp.sum(jnp.log2(g_ref[...]))
    o_ref[0, 0] = (jnp.float32(const_term)
                   + jnp.float32(-eps * ln2) * s_ref[0, 0]
                   + jnp.float32((eps - CONFIDENCE) * ln2) * s_tgt)


def kernel(x, target):
    n, size = x.shape
    eps = SMOOTHING / (size - 1)
    const_term = n * ((size - 1) * eps * math.log(eps)
                      + CONFIDENCE * math.log(CONFIDENCE))

    xt = x.T
    g = _sc_gather(xt, target, n)

    s_all = pl.pallas_call(
        _main_body,
        grid=(size // _BLK,),
        in_specs=[pl.BlockSpec((_BLK, n), lambda i: (i, 0))],
        out_specs=pl.BlockSpec(memory_space=pltpu.SMEM),
        out_shape=jax.ShapeDtypeStruct((1, 1), jnp.float32),
        compiler_params=pltpu.CompilerParams(
            dimension_semantics=("arbitrary",),
        ),
    )(xt)

    combine = functools.partial(_combine_body, const_term=const_term, eps=eps)
    out = pl.pallas_call(
        combine,
        in_specs=[
            pl.BlockSpec((8, n // 8), lambda: (0, 0)),
            pl.BlockSpec(memory_space=pltpu.SMEM),
        ],
        out_specs=pl.BlockSpec(memory_space=pltpu.SMEM),
        out_shape=jax.ShapeDtypeStruct((1, 1), jnp.float32),
    )(g.reshape(8, n // 8), s_all)
    return out[0, 0]


# speedup vs baseline: 1.4938x; 1.1515x over previous
import functools
import math

import jax
import jax.numpy as jnp
from jax.experimental import pallas as pl
from jax.experimental.pallas import tpu as pltpu

SMOOTHING = 0.1
CONFIDENCE = 1.0 - SMOOTHING
_BLK = 5000


def _body(xt_ref, t_ref, o_ref, *, const_term, eps, blk):
    i = pl.program_id(0)

    @pl.when(i == 0)
    def _():
        o_ref[0, 0] = jnp.float32(const_term)

    ln2 = math.log(2.0)
    logx = jnp.log2(xt_ref[...])
    row = i * blk + jax.lax.broadcasted_iota(jnp.int32, xt_ref.shape, 0)
    w = jnp.where(row == t_ref[...], jnp.float32(-CONFIDENCE * ln2),
                  jnp.float32(-eps * ln2))
    o_ref[0, 0] += jnp.sum(w * logx)


def kernel(x, target):
    n, size = x.shape
    eps = SMOOTHING / (size - 1)
    const_term = n * ((size - 1) * eps * math.log(eps)
                      + CONFIDENCE * math.log(CONFIDENCE))

    xt = x.T
    body = functools.partial(_body, const_term=const_term, eps=eps, blk=_BLK)
    out = pl.pallas_call(
        body,
        grid=(size // _BLK,),
        in_specs=[
            pl.BlockSpec((_BLK, n), lambda i: (i, 0)),
            pl.BlockSpec((1, n), lambda i: (0, 0)),
        ],
        out_specs=pl.BlockSpec(memory_space=pltpu.SMEM),
        out_shape=jax.ShapeDtypeStruct((1, 1), jnp.float32),
        compiler_params=pltpu.CompilerParams(
            dimension_semantics=("arbitrary",),
        ),
    )(xt, target.reshape(1, n))
    return out[0, 0]
